# Initial kernel scaffold; baseline (speedup 1.0000x reference)
import jax
import jax.numpy as jnp
from jax.experimental import pallas as pl


def kernel(x, edge_index, batch, Wl1, bl1, Wr1, Wl2, bl2, Wr2, Wl3, bl3, Wr3, Wp1, bp1, Wp2, bp2, Wp3, bp3, Wf1, bf1, Wf2, bf2, Wf3, bf3):
    raise NotImplementedError("write your pallas kernel here")


# speedup vs baseline: 1.9338x; 1.9338x over previous
import functools

import jax
import jax.numpy as jnp
from jax import lax
from jax.experimental import pallas as pl
from jax.experimental.pallas import tpu as pltpu
from jax.experimental.pallas import tpu_sc as plsc

F = 128
W = 144
G = 100
NC, NS, LN = 2, 16, 16
NW = NC * NS
E = 320000
BK = 80
EROWS = E // BK
NBLK = EROWS // NW
NEG = -3.4e38

f32 = jnp.float32
i32 = jnp.int32


def _mesh():
    return plsc.VectorSubcoreMesh(core_axis_name="c", subcore_axis_name="s")


def _sc_gs(w, ndp, name):
    rpt = ndp // NS

    @functools.partial(
        pl.kernel,
        out_type=jax.ShapeDtypeStruct((NC, ndp, w), f32),
        mesh=_mesh(),
        compiler_params=pltpu.CompilerParams(use_tc_tiling_on_sc=False, needs_layout_passes=False),
        scratch_types=[
            pltpu.VMEM_SHARED((ndp, w), f32),
            pltpu.VMEM((NBLK, BK), i32),
            pltpu.VMEM((NBLK, BK), i32),
            pltpu.VMEM((BK, w), f32),
            pltpu.VMEM((8, w), f32),
        ],
        name=name,
    )
    def k(src, nr, nc, out, acc, nrb, ncb, rows, zbuf):
        c = lax.axis_index("c")
        s = lax.axis_index("s")
        wid = c * NS + s
        for i in range(8):
            for q in range(w // LN):
                zbuf[i, pl.ds(q * LN, LN)] = jnp.zeros((LN,), f32)
        r0 = s * rpt

        @pl.loop(0, rpt // 8)
        def _(i):
            pltpu.sync_copy(zbuf, acc.at[pl.ds(r0 + i * 8, 8)])

        plsc.subcore_barrier()
        pltpu.sync_copy(nr.at[pl.ds(wid * NBLK, NBLK)], nrb)
        pltpu.sync_copy(nc.at[pl.ds(wid * NBLK, NBLK)], ncb)

        @pl.loop(0, NBLK)
        def _(j):
            pltpu.sync_copy(src.at[nrb.at[j]], rows)
            pltpu.sync_copy(rows, acc.at[ncb.at[j]], add=True)

        plsc.subcore_barrier()
        pltpu.sync_copy(acc.at[pl.ds(r0, rpt)], out.at[c, pl.ds(r0, rpt)])

    return k


def _sc_remap(nm_rows, trash, perm_rows, ndp, do_conv, name):
    w2 = W if do_conv else 16
    rpt = ndp // NS
    pb = perm_rows // NW

    @functools.partial(
        pl.kernel,
        out_type=(
            jax.ShapeDtypeStruct((EROWS, BK), i32),
            jax.ShapeDtypeStruct((EROWS, BK), i32),
            jax.ShapeDtypeStruct((perm_rows * BK, W), f32),
            jax.ShapeDtypeStruct((NC, ndp, w2), f32),
        ),
        mesh=_mesh(),
        compiler_params=pltpu.CompilerParams(use_tc_tiling_on_sc=False, needs_layout_passes=False),
        scratch_types=[
            pltpu.VMEM_SHARED((ndp, w2), f32),
            pltpu.VMEM((nm_rows,), i32),
            pltpu.VMEM((NBLK, BK), i32),
            pltpu.VMEM((NBLK, BK), i32),
            pltpu.VMEM((NBLK, BK), i32),
            pltpu.VMEM((NBLK, BK), i32),
            pltpu.VMEM((NBLK, BK), i32),
            pltpu.VMEM((BK, W), f32),
            pltpu.VMEM((8, w2), f32),
            pltpu.VMEM((BK, 16), f32),
            pltpu.VMEM((1, BK), i32),
        ],
        name=name,
    )
    def k(hg, nr, nc, nm, perm, nrn, ncn, xp, part,
          acc, nmb, nrb, ncb, nrob, ncob, xb, rows, zbuf, cb, pbuf):
        c = lax.axis_index("c")
        s = lax.axis_index("s")
        wid = c * NS + s
        for i in range(8):
            for q in range(w2 // LN):
                zbuf[i, pl.ds(q * LN, LN)] = jnp.zeros((LN,), f32)
        r0 = s * rpt

        @pl.loop(0, rpt // 8)
        def _(i):
            pltpu.sync_copy(zbuf, acc.at[pl.ds(r0 + i * 8, 8)])

        plsc.subcore_barrier()
        pltpu.sync_copy(nm, nmb)
        pltpu.sync_copy(nr.at[pl.ds(wid * NBLK, NBLK)], nrb)
        pltpu.sync_copy(nc.at[pl.ds(wid * NBLK, NBLK)], ncb)
        if not do_conv:
            oh = jnp.where(lax.broadcasted_iota(i32, (LN,), 0) == 0,
                           jnp.float32(1), jnp.float32(0))
            for i in range(BK):
                cb[i, pl.ds(0, 16)] = oh

        @pl.loop(0, NBLK)
        def _(j):
            for t in range(BK // LN):
                sl = pl.ds(t * LN, LN)
                a = plsc.load_gather(nmb, [nrb[j, sl]])
                b = plsc.load_gather(nmb, [ncb[j, sl]])
                valid = (a >= 0) & (b >= 0)
                nrob[j, sl] = jnp.where(valid, a, 0)
                if do_conv:
                    ncob[j, sl] = jnp.where(valid, b, trash)
                else:
                    ncob[j, sl] = jnp.where(valid, a, trash)
                    xb[j, sl] = jnp.where(valid, b, trash)

        pltpu.sync_copy(nrob, nrn.at[pl.ds(wid * NBLK, NBLK)])
        pltpu.sync_copy(ncob if do_conv else xb,
                        ncn.at[pl.ds(wid * NBLK, NBLK)])

        @pl.loop(0, NBLK)
        def _(j):
            if do_conv:
                pltpu.sync_copy(hg.at[nrb.at[j]], rows)
                pltpu.sync_copy(rows, acc.at[ncob.at[j]], add=True)
            else:
                pltpu.sync_copy(cb, acc.at[ncob.at[j]], add=True)

        plsc.subcore_barrier()
        pltpu.sync_copy(acc.at[pl.ds(r0, rpt)], part.at[c, pl.ds(r0, rpt)])

        for r in range(pb):
            pr = wid * pb + r
            pltpu.sync_copy(perm.at[pl.ds(pr, 1)], pbuf)
            pltpu.sync_copy(hg.at[pbuf.at[0]], rows)
            pltpu.sync_copy(rows, xp.at[pl.ds(pr * BK, BK)])

    return k


def _leaky(x):
    return jnp.where(x >= 0, x, 0.1 * x)


def _tc_dense(rc, br, name):
    def body(part, xf, wl, bl, wr, wp, h_o, u16_o, hh_o, dinv_o):
        sfull = part[0] + part[1]
        cnt = sfull[:, 128:129]
        aggr = sfull[:, :F] / jnp.maximum(cnt, 1.0)
        z = (jnp.dot(aggr, wl[...], preferred_element_type=f32)
             + bl[...]
             + jnp.dot(xf[...], wr[...], preferred_element_type=f32))
        h = _leaky(z)
        dinv = lax.rsqrt(cnt + 1.0)
        hh = jnp.dot(h, wp[...], preferred_element_type=f32)
        u = hh * dinv
        lane = lax.broadcasted_iota(i32, (br, 16), 1)
        h_o[...] = h
        u16_o[...] = jnp.where(lane == 0, u, 0.0)
        hh_o[...] = hh
        dinv_o[...] = dinv

    return pl.pallas_call(
        body,
        grid=(rc // br,),
        in_specs=[
            pl.BlockSpec((2, br, W), lambda i: (0, i, 0)),
            pl.BlockSpec((br, F), lambda i: (i, 0)),
            pl.BlockSpec((F, F), lambda i: (0, 0)),
            pl.BlockSpec((1, F), lambda i: (0, 0)),
            pl.BlockSpec((F, F), lambda i: (0, 0)),
            pl.BlockSpec((F, 1), lambda i: (0, 0)),
        ],
        out_specs=[
            pl.BlockSpec((br, F), lambda i: (i, 0)),
            pl.BlockSpec((br, 16), lambda i: (i, 0)),
            pl.BlockSpec((br, 1), lambda i: (i, 0)),
            pl.BlockSpec((br, 1), lambda i: (i, 0)),
        ],
        out_shape=[
            jax.ShapeDtypeStruct((rc, F), f32),
            jax.ShapeDtypeStruct((rc, 16), f32),
            jax.ShapeDtypeStruct((rc, 1), f32),
            jax.ShapeDtypeStruct((rc, 1), f32),
        ],
        name=name,
    )


def _tc_pool(n_per, kk, name):
    def body(spA, hh, dinv, h3d, bp, nm_o, perm_o, hg_o):
        sp = spA[0] + spA[1]
        d = dinv[...]
        score = sp * d + hh[...] * d * d + bp[0, 0]
        work = score
        iota = lax.broadcasted_iota(i32, (G, n_per), 1)
        rowid = lax.broadcasted_iota(i32, (G, n_per), 0)
        iota_k = lax.broadcasted_iota(i32, (G, kk), 1)
        rowid_k = lax.broadcasted_iota(i32, (G, kk), 0)
        nm2d = jnp.full((G, n_per), -1, i32)
        perm = jnp.zeros((G, kk), i32)
        for m in range(kk):
            mx = jnp.max(work, axis=1, keepdims=True)
            eq = work == mx
            jf = jnp.min(jnp.where(eq, iota, 1 << 30), axis=1, keepdims=True)
            oh = iota == jf
            nm2d = jnp.where(oh, rowid * kk + m, nm2d)
            perm = jnp.where(iota_k == m, jf + rowid_k * n_per, perm)
            work = jnp.where(oh, NEG, work)
        gate = jnp.tanh(score)
        nm_o[...] = nm2d
        perm_o[...] = perm
        hg_o[...] = h3d[...] * gate[:, :, None]

    return pl.pallas_call(
        body,
        out_shape=[
            jax.ShapeDtypeStruct((G, n_per), i32),
            jax.ShapeDtypeStruct((G, kk), i32),
            jax.ShapeDtypeStruct((G, n_per, F), f32),
        ],
        name=name,
    )


def _readout(xr, kk):
    mx = xr[:, 0, :]
    sm = xr[:, 0, :]
    for i in range(1, kk):
        v = xr[:, i, :]
        mx = jnp.maximum(mx, v)
        sm = sm + v
    return jnp.concatenate([mx, sm * (1.0 / kk)], axis=1)


def _tc_head(name):
    def body(x1r, x2r, x3r, degA, wf1, bf1, wf2, bf2, wf3, bf3,
             out_o, g_o, hd_o, dinv_o):
        x1h = _readout(x1r, 50)
        x2h = _readout(x2r, 25)
        x3h = _readout(x3r, 13)
        g = _leaky(x1h) + _leaky(x2h) + _leaky(x3h)
        o = _leaky(jnp.dot(g, wf1[...], preferred_element_type=f32) + bf1[...])
        o = _leaky(jnp.dot(o, wf2[...], preferred_element_type=f32) + bf2[...])
        o = jnp.dot(o, wf3[...], preferred_element_type=f32) + bf3[...]
        deg = degA[0] + degA[1]
        dinv0 = jnp.where(deg > 0,
                          lax.rsqrt(jnp.maximum(deg, 1e-12)), 0.0)
        out_o[...] = o
        g_o[...] = g
        hd_o[...] = x3r[...] * dinv0[:, :, None]
        dinv_o[...] = dinv0

    return pl.pallas_call(
        body,
        out_shape=[
            jax.ShapeDtypeStruct((G, 30), f32),
            jax.ShapeDtypeStruct((G, 256), f32),
            jax.ShapeDtypeStruct((G, 13, F), f32),
            jax.ShapeDtypeStruct((G, 13), f32),
        ],
        name=name,
    )


def _tc_info(name):
    def body(ip, x3f, dinv0, out_o):
        agg = (ip[0] + ip[1]) * dinv0[...]
        info = x3f[...] - agg
        ns = jnp.sum(jnp.abs(info), axis=1, keepdims=True)
        out_o[...] = jnp.sum(ns).reshape(1, 1) * (1.0 / 1300.0)

    return pl.pallas_call(
        body,
        out_shape=[jax.ShapeDtypeStruct((1, 1), f32)],
        name=name,
    )


def _const16(n):
    return jnp.concatenate([jnp.ones((n, 1), f32), jnp.zeros((n, 15), f32)],
                           axis=1)


def kernel(x, edge_index, batch, Wl1, bl1, Wr1, Wl2, bl2, Wr2, Wl3, bl3, Wr3,
           Wp1, bp1, Wp2, bp2, Wp3, bp3, Wf1, bf1, Wf2, bf2, Wf3, bf3):
    del batch
    nr0 = edge_index[0].reshape(EROWS, BK)
    nc0 = edge_index[1].reshape(EROWS, BK)
    hg0 = jnp.concatenate([x, _const16(10000)], axis=1)

    part1 = _sc_gs(W, 10112, "sc_conv1")(hg0, nr0, nc0)
    h1, u16_1, hh1, dinv1 = _tc_dense(10000, 1000, "tc_dense1")(
        part1[:, :10000], x, Wl1, bl1.reshape(1, F), Wr1, Wp1)
    sp1 = _sc_gs(16, 10112, "sc_score1")(u16_1, nr0, nc0)
    nm1_2d, perm1_2d, hg1_3d = _tc_pool(100, 50, "tc_pool1")(
        sp1[:, :10000, 0].reshape(2, G, 100),
        hh1[:, 0].reshape(G, 100),
        dinv1[:, 0].reshape(G, 100),
        h1.reshape(G, 100, F),
        bp1.reshape(1, 1))
    nm1 = nm1_2d.reshape(10000)
    perm1 = jnp.concatenate(
        [perm1_2d.reshape(5000), jnp.zeros((120,), i32)]).reshape(64, BK)
    hg1 = jnp.concatenate([hg1_3d.reshape(10000, F), _const16(10000)], axis=1)

    nr1, nc1, xp1, part2 = _sc_remap(
        10000, 5000, 64, 5120, True, "sc_remap_conv2")(hg1, nr0, nc0, nm1,
                                                       perm1)
    h2, u16_2, hh2, dinv2 = _tc_dense(5120, 640, "tc_dense2")(
        part2, xp1[:, :F], Wl2, bl2.reshape(1, F), Wr2, Wp2)
    sp2 = _sc_gs(16, 5120, "sc_score2")(u16_2, nr1, nc1)
    nm2_2d, perm2_2d, hg2_3d = _tc_pool(50, 25, "tc_pool2")(
        sp2[:, :5000, 0].reshape(2, G, 50),
        hh2[:5000, 0].reshape(G, 50),
        dinv2[:5000, 0].reshape(G, 50),
        h2[:5000].reshape(G, 50, F),
        bp2.reshape(1, 1))
    nm2 = jnp.concatenate([nm2_2d.reshape(5000), jnp.full((8,), -1, i32)])
    perm2 = jnp.concatenate(
        [perm2_2d.reshape(2500), jnp.zeros((60,), i32)]).reshape(32, BK)
    hg2 = jnp.concatenate([hg2_3d.reshape(5000, F), _const16(5000)], axis=1)

    nr2, nc2, xp2, part3 = _sc_remap(
        5008, 2500, 32, 2560, True, "sc_remap_conv3")(hg2, nr1, nc1, nm2,
                                                      perm2)
    h3, u16_3, hh3, dinv3 = _tc_dense(2560, 640, "tc_dense3")(
        part3, xp2[:, :F], Wl3, bl3.reshape(1, F), Wr3, Wp3)
    sp3 = _sc_gs(16, 2560, "sc_score3")(u16_3, nr2, nc2)
    nm3_2d, perm3_2d, hg3_3d = _tc_pool(25, 13, "tc_pool3")(
        sp3[:, :2500, 0].reshape(2, G, 25),
        hh3[:2500, 0].reshape(G, 25),
        dinv3[:2500, 0].reshape(G, 25),
        h3[:2500].reshape(G, 25, F),
        bp3.reshape(1, 1))
    nm3 = jnp.concatenate([nm3_2d.reshape(2500), jnp.full((4,), -1, i32)])
    perm3 = jnp.concatenate(
        [perm3_2d.reshape(1300), jnp.zeros((1260,), i32)]).reshape(32, BK)
    hg3 = jnp.concatenate([hg3_3d.reshape(2500, F), _const16(2500)], axis=1)

    nr3, nc3, xp3, degp = _sc_remap(
        2504, 1300, 32, 1408, False, "sc_remap_pool4")(hg3, nr2, nc2, nm3,
                                                       perm3)

    out, gemb, hd3, dinv0 = _tc_head("tc_head")(
        xp1[:5000, :F].reshape(G, 50, F),
        xp2[:2500, :F].reshape(G, 25, F),
        xp3[:1300, :F].reshape(G, 13, F),
        degp[:, :1300, 0].reshape(2, G, 13),
        Wf1, bf1.reshape(1, F), Wf2, bf2.reshape(1, 64), Wf3,
        bf3.reshape(1, 30))

    hd = jnp.concatenate([hd3.reshape(1300, F), _const16(1300)], axis=1)
    ipart = _sc_gs(W, 1408, "sc_info")(hd, nr3, nc3)
    (nsm,) = _tc_info("tc_info")(
        ipart[:, :1300, :F], xp3[:1300, :F], dinv0.reshape(1300, 1))

    return out, nsm.reshape(()), gemb


# speedup vs baseline: 25.6073x; 13.2421x over previous
import functools

import jax
import jax.numpy as jnp
from jax import lax
from jax.experimental import pallas as pl
from jax.experimental.pallas import tpu as pltpu
from jax.experimental.pallas import tpu_sc as plsc

F = 128
W = 144
G = 100
NC, NS, LN = 2, 16, 16
NW = NC * NS
E = 320000
BK = 80
EROWS = E // BK
NBLK = EROWS // NW
NEG = -3.4e38

f32 = jnp.float32
i32 = jnp.int32


def _mesh():
    return plsc.VectorSubcoreMesh(core_axis_name="c", subcore_axis_name="s")


def _sc_gs(w, ndp, name):
    rpt = ndp // NS

    @functools.partial(
        pl.kernel,
        out_type=jax.ShapeDtypeStruct((NC, ndp, w), f32),
        mesh=_mesh(),
        compiler_params=pltpu.CompilerParams(use_tc_tiling_on_sc=False, needs_layout_passes=False),
        scratch_types=[
            pltpu.VMEM_SHARED((ndp, w), f32),
            pltpu.VMEM((NBLK, BK), i32),
            pltpu.VMEM((NBLK, BK), i32),
            pltpu.VMEM((BK, w), f32),
            pltpu.VMEM((8, w), f32),
        ],
        name=name,
    )
    def k(src, nr, nc, out, acc, nrb, ncb, rows, zbuf):
        c = lax.axis_index("c")
        s = lax.axis_index("s")
        wid = c * NS + s
        for i in range(8):
            for q in range(w // LN):
                zbuf[i, pl.ds(q * LN, LN)] = jnp.zeros((LN,), f32)
        r0 = s * rpt

        @pl.loop(0, rpt // 8)
        def _(i):
            pltpu.sync_copy(zbuf, acc.at[pl.ds(r0 + i * 8, 8)])

        plsc.subcore_barrier()
        pltpu.sync_copy(nr.at[pl.ds(wid * NBLK, NBLK)], nrb)
        pltpu.sync_copy(nc.at[pl.ds(wid * NBLK, NBLK)], ncb)

        @pl.loop(0, NBLK)
        def _(j):
            pltpu.sync_copy(src.at[nrb.at[j]], rows)
            pltpu.sync_copy(rows, acc.at[ncb.at[j]], add=True)

        plsc.subcore_barrier()
        pltpu.sync_copy(acc.at[pl.ds(r0, rpt)], out.at[c, pl.ds(r0, rpt)])

    return k


def _sc_remap(nm_rows, trash, perm_rows, ndp, do_conv, name):
    w2 = W if do_conv else 16
    rpt = ndp // NS
    pb = perm_rows // NW

    @functools.partial(
        pl.kernel,
        out_type=(
            jax.ShapeDtypeStruct((EROWS, BK), i32),
            jax.ShapeDtypeStruct((EROWS, BK), i32),
            jax.ShapeDtypeStruct((perm_rows * BK, W), f32),
            jax.ShapeDtypeStruct((NC, ndp, w2), f32),
        ),
        mesh=_mesh(),
        compiler_params=pltpu.CompilerParams(use_tc_tiling_on_sc=False, needs_layout_passes=False),
        scratch_types=[
            pltpu.VMEM_SHARED((ndp, w2), f32),
            pltpu.VMEM((nm_rows,), i32),
            pltpu.VMEM((NBLK, BK), i32),
            pltpu.VMEM((NBLK, BK), i32),
            pltpu.VMEM((NBLK, BK), i32),
            pltpu.VMEM((NBLK, BK), i32),
            pltpu.VMEM((NBLK, BK), i32),
            pltpu.VMEM((BK, W), f32),
            pltpu.VMEM((8, w2), f32),
            pltpu.VMEM((BK, 16), f32),
            pltpu.VMEM((1, BK), i32),
        ],
        name=name,
    )
    def k(hg, nr, nc, nm, perm, nrn, ncn, xp, part,
          acc, nmb, nrb, ncb, nrob, ncob, xb, rows, zbuf, cb, pbuf):
        c = lax.axis_index("c")
        s = lax.axis_index("s")
        wid = c * NS + s
        for i in range(8):
            for q in range(w2 // LN):
                zbuf[i, pl.ds(q * LN, LN)] = jnp.zeros((LN,), f32)
        r0 = s * rpt

        @pl.loop(0, rpt // 8)
        def _(i):
            pltpu.sync_copy(zbuf, acc.at[pl.ds(r0 + i * 8, 8)])

        plsc.subcore_barrier()
        pltpu.sync_copy(nm, nmb)
        pltpu.sync_copy(nr.at[pl.ds(wid * NBLK, NBLK)], nrb)
        pltpu.sync_copy(nc.at[pl.ds(wid * NBLK, NBLK)], ncb)
        if not do_conv:
            oh = jnp.where(lax.broadcasted_iota(i32, (LN,), 0) == 0,
                           jnp.float32(1), jnp.float32(0))
            for i in range(BK):
                cb[i, pl.ds(0, 16)] = oh

        @pl.loop(0, NBLK)
        def _(j):
            for t in range(BK // LN):
                sl = pl.ds(t * LN, LN)
                nri = nrb[j, sl]
                a = plsc.load_gather(nmb, [nri])
                b = plsc.load_gather(nmb, [ncb[j, sl]])
                valid = (a >= 0) & (b >= 0)
                spread = jnp.where(nri >= trash, nri - trash, nri)
                nrob[j, sl] = jnp.where(valid, a, spread)
                if do_conv:
                    ncob[j, sl] = jnp.where(valid, b, trash)
                else:
                    ncob[j, sl] = jnp.where(valid, a, trash)
                    xb[j, sl] = jnp.where(valid, b, trash)

        pltpu.sync_copy(nrob, nrn.at[pl.ds(wid * NBLK, NBLK)])
        pltpu.sync_copy(ncob if do_conv else xb,
                        ncn.at[pl.ds(wid * NBLK, NBLK)])

        @pl.loop(0, NBLK)
        def _(j):
            if do_conv:
                pltpu.sync_copy(hg.at[nrb.at[j]], rows)
                pltpu.sync_copy(rows, acc.at[ncob.at[j]], add=True)
            else:
                pltpu.sync_copy(cb, acc.at[ncob.at[j]], add=True)

        plsc.subcore_barrier()
        pltpu.sync_copy(acc.at[pl.ds(r0, rpt)], part.at[c, pl.ds(r0, rpt)])

        for r in range(pb):
            pr = wid * pb + r
            pltpu.sync_copy(perm.at[pl.ds(pr, 1)], pbuf)
            pltpu.sync_copy(hg.at[pbuf.at[0]], rows)
            pltpu.sync_copy(rows, xp.at[pl.ds(pr * BK, BK)])

    return k


def _leaky(x):
    return jnp.where(x >= 0, x, 0.1 * x)


def _tc_dense(rc, br, name):
    def body(part, xf, wl, bl, wr, wp, h_o, u16_o, hh_o, dinv_o):
        sfull = part[0] + part[1]
        cnt = sfull[:, 128:129]
        aggr = sfull[:, :F] / jnp.maximum(cnt, 1.0)
        z = (jnp.dot(aggr, wl[...], preferred_element_type=f32)
             + bl[...]
             + jnp.dot(xf[...], wr[...], preferred_element_type=f32))
        h = _leaky(z)
        dinv = lax.rsqrt(cnt + 1.0)
        hh = jnp.dot(h, wp[...], preferred_element_type=f32)
        u = hh * dinv
        lane = lax.broadcasted_iota(i32, (br, 16), 1)
        h_o[...] = h
        u16_o[...] = jnp.where(lane == 0, u, 0.0)
        hh_o[...] = hh
        dinv_o[...] = dinv

    return pl.pallas_call(
        body,
        grid=(rc // br,),
        in_specs=[
            pl.BlockSpec((2, br, W), lambda i: (0, i, 0)),
            pl.BlockSpec((br, F), lambda i: (i, 0)),
            pl.BlockSpec((F, F), lambda i: (0, 0)),
            pl.BlockSpec((1, F), lambda i: (0, 0)),
            pl.BlockSpec((F, F), lambda i: (0, 0)),
            pl.BlockSpec((F, 1), lambda i: (0, 0)),
        ],
        out_specs=[
            pl.BlockSpec((br, F), lambda i: (i, 0)),
            pl.BlockSpec((br, 16), lambda i: (i, 0)),
            pl.BlockSpec((br, 1), lambda i: (i, 0)),
            pl.BlockSpec((br, 1), lambda i: (i, 0)),
        ],
        out_shape=[
            jax.ShapeDtypeStruct((rc, F), f32),
            jax.ShapeDtypeStruct((rc, 16), f32),
            jax.ShapeDtypeStruct((rc, 1), f32),
            jax.ShapeDtypeStruct((rc, 1), f32),
        ],
        name=name,
    )


def _tc_pool(n_per, kk, name):
    def body(spA, hh, dinv, h3d, bp, nm_o, perm_o, hg_o):
        sp = spA[0] + spA[1]
        d = dinv[...]
        score = sp * d + hh[...] * d * d + bp[0, 0]
        work = score
        iota = lax.broadcasted_iota(i32, (G, n_per), 1)
        rowid = lax.broadcasted_iota(i32, (G, n_per), 0)
        iota_k = lax.broadcasted_iota(i32, (G, kk), 1)
        rowid_k = lax.broadcasted_iota(i32, (G, kk), 0)
        nm2d = jnp.full((G, n_per), -1, i32)
        perm = jnp.zeros((G, kk), i32)
        for m in range(kk):
            mx = jnp.max(work, axis=1, keepdims=True)
            eq = work == mx
            jf = jnp.min(jnp.where(eq, iota, 1 << 30), axis=1, keepdims=True)
            oh = iota == jf
            nm2d = jnp.where(oh, rowid * kk + m, nm2d)
            perm = jnp.where(iota_k == m, jf + rowid_k * n_per, perm)
            work = jnp.where(oh, NEG, work)
        gate = jnp.tanh(score)
        nm_o[...] = nm2d
        perm_o[...] = perm
        hg_o[...] = h3d[...] * gate[:, :, None]

    return pl.pallas_call(
        body,
        out_shape=[
            jax.ShapeDtypeStruct((G, n_per), i32),
            jax.ShapeDtypeStruct((G, kk), i32),
            jax.ShapeDtypeStruct((G, n_per, F), f32),
        ],
        name=name,
    )


def _readout(xr, kk):
    mx = xr[:, 0, :]
    sm = xr[:, 0, :]
    for i in range(1, kk):
        v = xr[:, i, :]
        mx = jnp.maximum(mx, v)
        sm = sm + v
    return jnp.concatenate([mx, sm * (1.0 / kk)], axis=1)


def _tc_head(name):
    def body(x1r, x2r, x3r, degA, wf1, bf1, wf2, bf2, wf3, bf3,
             out_o, g_o, hd_o, dinv_o):
        x1h = _readout(x1r, 50)
        x2h = _readout(x2r, 25)
        x3h = _readout(x3r, 13)
        g = _leaky(x1h) + _leaky(x2h) + _leaky(x3h)
        o = _leaky(jnp.dot(g, wf1[...], preferred_element_type=f32) + bf1[...])
        o = _leaky(jnp.dot(o, wf2[...], preferred_element_type=f32) + bf2[...])
        o = jnp.dot(o, wf3[...], preferred_element_type=f32) + bf3[...]
        deg = degA[0] + degA[1]
        dinv0 = jnp.where(deg > 0,
                          lax.rsqrt(jnp.maximum(deg, 1e-12)), 0.0)
        out_o[...] = o
        g_o[...] = g
        hd_o[...] = x3r[...] * dinv0[:, :, None]
        dinv_o[...] = dinv0

    return pl.pallas_call(
        body,
        out_shape=[
            jax.ShapeDtypeStruct((G, 30), f32),
            jax.ShapeDtypeStruct((G, 256), f32),
            jax.ShapeDtypeStruct((G, 13, F), f32),
            jax.ShapeDtypeStruct((G, 13), f32),
        ],
        name=name,
    )


def _tc_info(name):
    def body(ip, x3f, dinv0, out_o):
        agg = (ip[0] + ip[1]) * dinv0[...]
        info = x3f[...] - agg
        ns = jnp.sum(jnp.abs(info), axis=1, keepdims=True)
        out_o[...] = jnp.sum(ns).reshape(1, 1) * (1.0 / 1300.0)

    return pl.pallas_call(
        body,
        out_shape=[jax.ShapeDtypeStruct((1, 1), f32)],
        name=name,
    )


def _const16(n):
    return jnp.concatenate([jnp.ones((n, 1), f32), jnp.zeros((n, 15), f32)],
                           axis=1)


def kernel(x, edge_index, batch, Wl1, bl1, Wr1, Wl2, bl2, Wr2, Wl3, bl3, Wr3,
           Wp1, bp1, Wp2, bp2, Wp3, bp3, Wf1, bf1, Wf2, bf2, Wf3, bf3):
    del batch
    nr0 = edge_index[0].reshape(EROWS, BK)
    nc0 = edge_index[1].reshape(EROWS, BK)
    hg0 = jnp.concatenate([x, _const16(10000)], axis=1)

    part1 = _sc_gs(W, 10112, "sc_conv1")(hg0, nr0, nc0)
    h1, u16_1, hh1, dinv1 = _tc_dense(10000, 1000, "tc_dense1")(
        part1[:, :10000], x, Wl1, bl1.reshape(1, F), Wr1, Wp1)
    sp1 = _sc_gs(16, 10112, "sc_score1")(u16_1, nr0, nc0)
    nm1_2d, perm1_2d, hg1_3d = _tc_pool(100, 50, "tc_pool1")(
        sp1[:, :10000, 0].reshape(2, G, 100),
        hh1[:, 0].reshape(G, 100),
        dinv1[:, 0].reshape(G, 100),
        h1.reshape(G, 100, F),
        bp1.reshape(1, 1))
    nm1 = nm1_2d.reshape(10000)
    perm1 = jnp.concatenate(
        [perm1_2d.reshape(5000), jnp.zeros((120,), i32)]).reshape(64, BK)
    hg1 = jnp.concatenate([hg1_3d.reshape(10000, F), _const16(10000)], axis=1)

    nr1, nc1, xp1, part2 = _sc_remap(
        10000, 5000, 64, 5120, True, "sc_remap_conv2")(hg1, nr0, nc0, nm1,
                                                       perm1)
    h2, u16_2, hh2, dinv2 = _tc_dense(5120, 640, "tc_dense2")(
        part2, xp1[:, :F], Wl2, bl2.reshape(1, F), Wr2, Wp2)
    sp2 = _sc_gs(16, 5120, "sc_score2")(u16_2, nr1, nc1)
    nm2_2d, perm2_2d, hg2_3d = _tc_pool(50, 25, "tc_pool2")(
        sp2[:, :5000, 0].reshape(2, G, 50),
        hh2[:5000, 0].reshape(G, 50),
        dinv2[:5000, 0].reshape(G, 50),
        h2[:5000].reshape(G, 50, F),
        bp2.reshape(1, 1))
    nm2 = jnp.concatenate([nm2_2d.reshape(5000), jnp.full((8,), -1, i32)])
    perm2 = jnp.concatenate(
        [perm2_2d.reshape(2500), jnp.zeros((60,), i32)]).reshape(32, BK)
    hg2 = jnp.concatenate([hg2_3d.reshape(5000, F), _const16(5000)], axis=1)

    nr2, nc2, xp2, part3 = _sc_remap(
        5008, 2500, 32, 2560, True, "sc_remap_conv3")(hg2, nr1, nc1, nm2,
                                                      perm2)
    h3, u16_3, hh3, dinv3 = _tc_dense(2560, 640, "tc_dense3")(
        part3, xp2[:, :F], Wl3, bl3.reshape(1, F), Wr3, Wp3)
    sp3 = _sc_gs(16, 2560, "sc_score3")(u16_3, nr2, nc2)
    nm3_2d, perm3_2d, hg3_3d = _tc_pool(25, 13, "tc_pool3")(
        sp3[:, :2500, 0].reshape(2, G, 25),
        hh3[:2500, 0].reshape(G, 25),
        dinv3[:2500, 0].reshape(G, 25),
        h3[:2500].reshape(G, 25, F),
        bp3.reshape(1, 1))
    nm3 = jnp.concatenate([nm3_2d.reshape(2500), jnp.full((4,), -1, i32)])
    perm3 = jnp.concatenate(
        [perm3_2d.reshape(1300), jnp.zeros((1260,), i32)]).reshape(32, BK)
    hg3 = jnp.concatenate([hg3_3d.reshape(2500, F), _const16(2500)], axis=1)

    nr3, nc3, xp3, degp = _sc_remap(
        2504, 1300, 32, 1408, False, "sc_remap_pool4")(hg3, nr2, nc2, nm3,
                                                       perm3)

    out, gemb, hd3, dinv0 = _tc_head("tc_head")(
        xp1[:5000, :F].reshape(G, 50, F),
        xp2[:2500, :F].reshape(G, 25, F),
        xp3[:1300, :F].reshape(G, 13, F),
        degp[:, :1300, 0].reshape(2, G, 13),
        Wf1, bf1.reshape(1, F), Wf2, bf2.reshape(1, 64), Wf3,
        bf3.reshape(1, 30))

    hd = jnp.concatenate([hd3.reshape(1300, F), _const16(1300)], axis=1)
    ipart = _sc_gs(W, 1408, "sc_info")(hd, nr3, nc3)
    (nsm,) = _tc_info("tc_info")(
        ipart[:, :1300, :F], xp3[:1300, :F], dinv0.reshape(1300, 1))

    return out, nsm.reshape(()), gemb


# speedup vs baseline: 28.0923x; 1.0970x over previous
import functools

import jax
import jax.numpy as jnp
from jax import lax
from jax.experimental import pallas as pl
from jax.experimental.pallas import tpu as pltpu
from jax.experimental.pallas import tpu_sc as plsc

F = 128
W = 144
G = 100
NC, NS, LN = 2, 16, 16
NW = NC * NS
E = 320000
BK = 80
EROWS = E // BK
NBLK = EROWS // NW
NBUF = 5
NEG = -3.4e38

f32 = jnp.float32
i32 = jnp.int32


def _mesh():
    return plsc.VectorSubcoreMesh(core_axis_name="c", subcore_axis_name="s")


def _ring(nblk, nbuf, bufs, g_start, g_wait, s_start, s_wait):
    g_start(0, 0)
    ngrp = -(-nblk // nbuf)

    @pl.loop(0, ngrp)
    def _(g):
        j0 = g * nbuf
        for u in range(nbuf):
            j = j0 + u

            @pl.when((j >= nbuf - 1) & (j < nblk))
            def _():
                s_wait((u + 1) % nbuf)

            @pl.when(j < nblk - 1)
            def _():
                g_start(j + 1, (u + 1) % nbuf)

            @pl.when(j < nblk)
            def _():
                g_wait(u)
                s_start(j, u)

    for d in range(nbuf - 1):
        s_wait((nblk - (nbuf - 1) + d) % nbuf)


def _sc_gs(w, ndp, nbuf, bk, name):
    rpt = ndp // NS
    erows = E // bk
    nblk = erows // NW

    @functools.partial(
        pl.kernel,
        out_type=jax.ShapeDtypeStruct((NC, ndp, w), f32),
        mesh=_mesh(),
        compiler_params=pltpu.CompilerParams(use_tc_tiling_on_sc=False, needs_layout_passes=False),
        scratch_types=[
            pltpu.VMEM_SHARED((ndp, w), f32),
            pltpu.VMEM((nblk, bk), i32),
            pltpu.VMEM((nblk, bk), i32),
        ] + [pltpu.VMEM((bk, w), f32) for _ in range(nbuf)] + [
            pltpu.VMEM((8, w), f32),
            pltpu.SemaphoreType.DMA,
            pltpu.SemaphoreType.DMA,
        ],
        name=name,
    )
    def k(src, nr, nc, out, *scr):
        acc, nrb, ncb = scr[0], scr[1], scr[2]
        bufs = scr[3:3 + nbuf]
        zbuf, gsem, ssem = scr[3 + nbuf], scr[4 + nbuf], scr[5 + nbuf]
        c = lax.axis_index("c")
        s = lax.axis_index("s")
        wid = c * NS + s
        pltpu.async_copy(nr.at[pl.ds(wid * nblk, nblk)], nrb, gsem)
        pltpu.async_copy(nc.at[pl.ds(wid * nblk, nblk)], ncb, gsem)
        for i in range(8):
            for q in range(w // LN):
                zbuf[i, pl.ds(q * LN, LN)] = jnp.zeros((LN,), f32)
        r0 = s * rpt

        @pl.loop(0, rpt // 8)
        def _(i):
            pltpu.async_copy(zbuf, acc.at[pl.ds(r0 + i * 8, 8)], ssem)

        @pl.loop(0, rpt // 8)
        def _(i):
            pltpu.make_async_copy(zbuf, acc.at[pl.ds(r0, 8)], ssem).wait()

        pltpu.make_async_copy(nr.at[pl.ds(0, nblk)], nrb, gsem).wait()
        pltpu.make_async_copy(nc.at[pl.ds(0, nblk)], ncb, gsem).wait()
        plsc.subcore_barrier()

        def g_start(j, u):
            pltpu.async_copy(src.at[nrb.at[j]], bufs[u], gsem)

        def g_wait(u):
            pltpu.make_async_copy(src.at[nrb.at[0]], bufs[u], gsem).wait()

        def s_start(j, u):
            pltpu.async_copy(bufs[u], acc.at[ncb.at[j]], ssem, add=True)

        def s_wait(u):
            pltpu.make_async_copy(bufs[u], acc.at[ncb.at[0]], ssem).wait()

        _ring(nblk, nbuf, bufs, g_start, g_wait, s_start, s_wait)
        plsc.subcore_barrier()
        pltpu.sync_copy(acc.at[pl.ds(r0, rpt)], out.at[c, pl.ds(r0, rpt)])

    return k


def _sc_remap(nm_rows, trash, perm_rows, ndp, do_conv, nbuf, name):
    w2 = W if do_conv else 16
    rpt = ndp // NS
    pb = perm_rows // NW

    scratch = [
        pltpu.VMEM_SHARED((ndp, w2), f32),
        pltpu.VMEM((nm_rows,), i32),
        pltpu.VMEM((NBLK, BK), i32),
        pltpu.VMEM((NBLK, BK), i32),
        pltpu.VMEM((NBLK, BK), i32),
        pltpu.VMEM((NBLK, BK), i32),
        pltpu.VMEM((8, 8) if do_conv else (NBLK, BK), i32),
    ] + [pltpu.VMEM((BK, W if do_conv else 16), f32) for _ in range(nbuf)] + [
        pltpu.VMEM((8, 8) if do_conv else (BK, W), f32),
        pltpu.VMEM((8, w2), f32),
        pltpu.VMEM((1, BK), i32),
        pltpu.SemaphoreType.DMA,
        pltpu.SemaphoreType.DMA,
    ]

    @functools.partial(
        pl.kernel,
        out_type=(
            jax.ShapeDtypeStruct((EROWS, BK), i32),
            jax.ShapeDtypeStruct((EROWS, BK), i32),
            jax.ShapeDtypeStruct((perm_rows * BK, W), f32),
            jax.ShapeDtypeStruct((NC, ndp, w2), f32),
        ),
        mesh=_mesh(),
        compiler_params=pltpu.CompilerParams(use_tc_tiling_on_sc=False, needs_layout_passes=False),
        scratch_types=scratch,
        name=name,
    )
    def k(hg, nr, nc, nm, perm, nrn, ncn, xp, part, *scr):
        acc, nmb, nrb, ncb, nrob, ncob, xb = scr[:7]
        bufs = scr[7:7 + nbuf]
        prow, zbuf, pbuf, gsem, ssem = scr[7 + nbuf:12 + nbuf]
        c = lax.axis_index("c")
        s = lax.axis_index("s")
        wid = c * NS + s
        pltpu.async_copy(nm, nmb, gsem)
        pltpu.async_copy(nr.at[pl.ds(wid * NBLK, NBLK)], nrb, gsem)
        pltpu.async_copy(nc.at[pl.ds(wid * NBLK, NBLK)], ncb, gsem)
        for i in range(8):
            for q in range(w2 // LN):
                zbuf[i, pl.ds(q * LN, LN)] = jnp.zeros((LN,), f32)
        r0 = s * rpt

        @pl.loop(0, rpt // 8)
        def _(i):
            pltpu.async_copy(zbuf, acc.at[pl.ds(r0 + i * 8, 8)], ssem)

        @pl.loop(0, rpt // 8)
        def _(i):
            pltpu.make_async_copy(zbuf, acc.at[pl.ds(r0, 8)], ssem).wait()

        pltpu.make_async_copy(nm, nmb, gsem).wait()
        pltpu.make_async_copy(nr.at[pl.ds(0, NBLK)], nrb, gsem).wait()
        pltpu.make_async_copy(nc.at[pl.ds(0, NBLK)], ncb, gsem).wait()
        if not do_conv:
            oh = jnp.where(lax.broadcasted_iota(i32, (LN,), 0) == 0,
                           jnp.float32(1), jnp.float32(0))
            for i in range(BK):
                bufs[0][i, pl.ds(0, 16)] = oh

        @pl.loop(0, NBLK)
        def _(j):
            for t in range(BK // LN):
                sl = pl.ds(t * LN, LN)
                nri = nrb[j, sl]
                a = plsc.load_gather(nmb, [nri])
                b = plsc.load_gather(nmb, [ncb[j, sl]])
                valid = (a >= 0) & (b >= 0)
                spread = jnp.where(nri >= trash, nri - trash, nri)
                nrob[j, sl] = jnp.where(valid, a, spread)
                if do_conv:
                    ncob[j, sl] = jnp.where(valid, b, trash)
                else:
                    ncob[j, sl] = jnp.where(valid, a, trash)
                    xb[j, sl] = jnp.where(valid, b, trash)

        pltpu.async_copy(nrob, nrn.at[pl.ds(wid * NBLK, NBLK)], gsem)
        pltpu.async_copy(ncob if do_conv else xb,
                         ncn.at[pl.ds(wid * NBLK, NBLK)], gsem)
        pltpu.make_async_copy(nrob, nrn.at[pl.ds(0, NBLK)], gsem).wait()
        pltpu.make_async_copy(ncob, ncn.at[pl.ds(0, NBLK)], gsem).wait()
        plsc.subcore_barrier()

        def g_start(j, u):
            pltpu.async_copy(hg.at[nrb.at[j]], bufs[u], gsem)

        def g_wait(u):
            pltpu.make_async_copy(hg.at[nrb.at[0]], bufs[u], gsem).wait()

        def s_start(j, u):
            pltpu.async_copy(bufs[u], acc.at[ncob.at[j]], ssem, add=True)

        def s_wait(u):
            pltpu.make_async_copy(bufs[u], acc.at[ncob.at[0]], ssem).wait()

        if do_conv:
            _ring(NBLK, nbuf, bufs, g_start, g_wait, s_start, s_wait)
        else:
            @pl.loop(0, NBLK)
            def _(j):
                @pl.when(j >= nbuf - 1)
                def _():
                    pltpu.make_async_copy(
                        bufs[0], acc.at[ncob.at[0]], ssem).wait()

                pltpu.async_copy(bufs[0], acc.at[ncob.at[j]], ssem,
                                 add=True)

            for d in range(nbuf - 1):
                pltpu.make_async_copy(bufs[0], acc.at[ncob.at[0]],
                                      ssem).wait()

        plsc.subcore_barrier()
        pltpu.sync_copy(acc.at[pl.ds(r0, rpt)], part.at[c, pl.ds(r0, rpt)])

        prow_ref = bufs[0] if do_conv else prow
        for r in range(pb):
            pr = wid * pb + r
            pltpu.sync_copy(perm.at[pl.ds(pr, 1)], pbuf)
            pltpu.sync_copy(hg.at[pbuf.at[0]], prow_ref)
            pltpu.sync_copy(prow_ref, xp.at[pl.ds(pr * BK, BK)])

    return k


def _leaky(x):
    return jnp.where(x >= 0, x, 0.1 * x)


def _tc_dense(rc, br, name):
    def body(part, xf, wl, bl, wr, wp, h_o, u16_o, hh_o, dinv_o):
        sfull = part[0] + part[1]
        cnt = sfull[:, 128:129]
        aggr = sfull[:, :F] / jnp.maximum(cnt, 1.0)
        z = (jnp.dot(aggr, wl[...], preferred_element_type=f32)
             + bl[...]
             + jnp.dot(xf[...], wr[...], preferred_element_type=f32))
        h = _leaky(z)
        dinv = lax.rsqrt(cnt + 1.0)
        hh = jnp.dot(h, wp[...], preferred_element_type=f32)
        u = hh * dinv
        lane = lax.broadcasted_iota(i32, (br, 16), 1)
        h_o[...] = h
        u16_o[...] = jnp.where(lane == 0, u, 0.0)
        hh_o[...] = hh
        dinv_o[...] = dinv

    return pl.pallas_call(
        body,
        grid=(rc // br,),
        in_specs=[
            pl.BlockSpec((2, br, W), lambda i: (0, i, 0)),
            pl.BlockSpec((br, F), lambda i: (i, 0)),
            pl.BlockSpec((F, F), lambda i: (0, 0)),
            pl.BlockSpec((1, F), lambda i: (0, 0)),
            pl.BlockSpec((F, F), lambda i: (0, 0)),
            pl.BlockSpec((F, 1), lambda i: (0, 0)),
        ],
        out_specs=[
            pl.BlockSpec((br, F), lambda i: (i, 0)),
            pl.BlockSpec((br, 16), lambda i: (i, 0)),
            pl.BlockSpec((br, 1), lambda i: (i, 0)),
            pl.BlockSpec((br, 1), lambda i: (i, 0)),
        ],
        out_shape=[
            jax.ShapeDtypeStruct((rc, F), f32),
            jax.ShapeDtypeStruct((rc, 16), f32),
            jax.ShapeDtypeStruct((rc, 1), f32),
            jax.ShapeDtypeStruct((rc, 1), f32),
        ],
        name=name,
    )


def _tc_pool(n_per, kk, name):
    def body(spA, hh, dinv, h3d, bp, nm_o, perm_o, hg_o):
        sp = spA[0] + spA[1]
        d = dinv[...]
        score = sp * d + hh[...] * d * d + bp[0, 0]
        work = score
        iota = lax.broadcasted_iota(i32, (G, n_per), 1)
        rowid = lax.broadcasted_iota(i32, (G, n_per), 0)
        iota_k = lax.broadcasted_iota(i32, (G, kk), 1)
        rowid_k = lax.broadcasted_iota(i32, (G, kk), 0)
        nm2d = jnp.full((G, n_per), -1, i32)
        perm = jnp.zeros((G, kk), i32)
        for m in range(kk):
            mx = jnp.max(work, axis=1, keepdims=True)
            eq = work == mx
            jf = jnp.min(jnp.where(eq, iota, 1 << 30), axis=1, keepdims=True)
            oh = iota == jf
            nm2d = jnp.where(oh, rowid * kk + m, nm2d)
            perm = jnp.where(iota_k == m, jf + rowid_k * n_per, perm)
            work = jnp.where(oh, NEG, work)
        gate = jnp.tanh(score)
        nm_o[...] = nm2d
        perm_o[...] = perm
        hg_o[...] = h3d[...] * gate[:, :, None]

    return pl.pallas_call(
        body,
        out_shape=[
            jax.ShapeDtypeStruct((G, n_per), i32),
            jax.ShapeDtypeStruct((G, kk), i32),
            jax.ShapeDtypeStruct((G, n_per, F), f32),
        ],
        name=name,
    )


def _readout(xr, kk):
    mx = xr[:, 0, :]
    sm = xr[:, 0, :]
    for i in range(1, kk):
        v = xr[:, i, :]
        mx = jnp.maximum(mx, v)
        sm = sm + v
    return jnp.concatenate([mx, sm * (1.0 / kk)], axis=1)


def _tc_head(name):
    def body(x1r, x2r, x3r, degA, wf1, bf1, wf2, bf2, wf3, bf3,
             out_o, g_o, hd_o, dinv_o):
        x1h = _readout(x1r, 50)
        x2h = _readout(x2r, 25)
        x3h = _readout(x3r, 13)
        g = _leaky(x1h) + _leaky(x2h) + _leaky(x3h)
        o = _leaky(jnp.dot(g, wf1[...], preferred_element_type=f32) + bf1[...])
        o = _leaky(jnp.dot(o, wf2[...], preferred_element_type=f32) + bf2[...])
        o = jnp.dot(o, wf3[...], preferred_element_type=f32) + bf3[...]
        deg = degA[0] + degA[1]
        dinv0 = jnp.where(deg > 0,
                          lax.rsqrt(jnp.maximum(deg, 1e-12)), 0.0)
        out_o[...] = o
        g_o[...] = g
        hd_o[...] = x3r[...] * dinv0[:, :, None]
        dinv_o[...] = dinv0

    return pl.pallas_call(
        body,
        out_shape=[
            jax.ShapeDtypeStruct((G, 30), f32),
            jax.ShapeDtypeStruct((G, 256), f32),
            jax.ShapeDtypeStruct((G, 13, F), f32),
            jax.ShapeDtypeStruct((G, 13), f32),
        ],
        name=name,
    )


def _tc_info(name):
    def body(ip, x3f, dinv0, out_o):
        agg = (ip[0] + ip[1]) * dinv0[...]
        info = x3f[...] - agg
        ns = jnp.sum(jnp.abs(info), axis=1, keepdims=True)
        out_o[...] = jnp.sum(ns).reshape(1, 1) * (1.0 / 1300.0)

    return pl.pallas_call(
        body,
        out_shape=[jax.ShapeDtypeStruct((1, 1), f32)],
        name=name,
    )


def _const16(n):
    return jnp.concatenate([jnp.ones((n, 1), f32), jnp.zeros((n, 15), f32)],
                           axis=1)


def kernel(x, edge_index, batch, Wl1, bl1, Wr1, Wl2, bl2, Wr2, Wl3, bl3, Wr3,
           Wp1, bp1, Wp2, bp2, Wp3, bp3, Wf1, bf1, Wf2, bf2, Wf3, bf3):
    del batch
    nr0 = edge_index[0].reshape(EROWS, BK)
    nc0 = edge_index[1].reshape(EROWS, BK)
    nr0c = edge_index[0].reshape(E // 50, 50)
    nc0c = edge_index[1].reshape(E // 50, 50)
    hg0 = jnp.concatenate([x, _const16(10000)], axis=1)

    part1 = _sc_gs(W, 10112, 2, 50, "sc_conv1")(hg0, nr0c, nc0c)
    h1, u16_1, hh1, dinv1 = _tc_dense(10000, 1000, "tc_dense1")(
        part1[:, :10000], x, Wl1, bl1.reshape(1, F), Wr1, Wp1)
    sp1 = _sc_gs(16, 10112, 5, 80, "sc_score1")(u16_1, nr0, nc0)
    nm1_2d, perm1_2d, hg1_3d = _tc_pool(100, 50, "tc_pool1")(
        sp1[:, :10000, 0].reshape(2, G, 100),
        hh1[:, 0].reshape(G, 100),
        dinv1[:, 0].reshape(G, 100),
        h1.reshape(G, 100, F),
        bp1.reshape(1, 1))
    nm1 = nm1_2d.reshape(10000)
    perm1 = jnp.concatenate(
        [perm1_2d.reshape(5000), jnp.zeros((120,), i32)]).reshape(64, BK)
    hg1 = jnp.concatenate([hg1_3d.reshape(10000, F), _const16(10000)], axis=1)

    nr1, nc1, xp1, part2 = _sc_remap(
        10000, 5000, 64, 5120, True, 2, "sc_remap_conv2")(hg1, nr0, nc0, nm1,
                                                       perm1)
    h2, u16_2, hh2, dinv2 = _tc_dense(5120, 640, "tc_dense2")(
        part2, xp1[:, :F], Wl2, bl2.reshape(1, F), Wr2, Wp2)
    sp2 = _sc_gs(16, 5120, 5, 80, "sc_score2")(u16_2, nr1, nc1)
    nm2_2d, perm2_2d, hg2_3d = _tc_pool(50, 25, "tc_pool2")(
        sp2[:, :5000, 0].reshape(2, G, 50),
        hh2[:5000, 0].reshape(G, 50),
        dinv2[:5000, 0].reshape(G, 50),
        h2[:5000].reshape(G, 50, F),
        bp2.reshape(1, 1))
    nm2 = jnp.concatenate([nm2_2d.reshape(5000), jnp.full((8,), -1, i32)])
    perm2 = jnp.concatenate(
        [perm2_2d.reshape(2500), jnp.zeros((60,), i32)]).reshape(32, BK)
    hg2 = jnp.concatenate([hg2_3d.reshape(5000, F), _const16(5000)], axis=1)

    nr2, nc2, xp2, part3 = _sc_remap(
        5008, 2500, 32, 2560, True, 5, "sc_remap_conv3")(hg2, nr1, nc1, nm2,
                                                      perm2)
    h3, u16_3, hh3, dinv3 = _tc_dense(2560, 640, "tc_dense3")(
        part3, xp2[:, :F], Wl3, bl3.reshape(1, F), Wr3, Wp3)
    sp3 = _sc_gs(16, 2560, 5, 80, "sc_score3")(u16_3, nr2, nc2)
    nm3_2d, perm3_2d, hg3_3d = _tc_pool(25, 13, "tc_pool3")(
        sp3[:, :2500, 0].reshape(2, G, 25),
        hh3[:2500, 0].reshape(G, 25),
        dinv3[:2500, 0].reshape(G, 25),
        h3[:2500].reshape(G, 25, F),
        bp3.reshape(1, 1))
    nm3 = jnp.concatenate([nm3_2d.reshape(2500), jnp.full((4,), -1, i32)])
    perm3 = jnp.concatenate(
        [perm3_2d.reshape(1300), jnp.zeros((1260,), i32)]).reshape(32, BK)
    hg3 = jnp.concatenate([hg3_3d.reshape(2500, F), _const16(2500)], axis=1)

    nr3, nc3, xp3, degp = _sc_remap(
        2504, 1300, 32, 1408, False, 5, "sc_remap_pool4")(hg3, nr2, nc2, nm3,
                                                       perm3)

    out, gemb, hd3, dinv0 = _tc_head("tc_head")(
        xp1[:5000, :F].reshape(G, 50, F),
        xp2[:2500, :F].reshape(G, 25, F),
        xp3[:1300, :F].reshape(G, 13, F),
        degp[:, :1300, 0].reshape(2, G, 13),
        Wf1, bf1.reshape(1, F), Wf2, bf2.reshape(1, 64), Wf3,
        bf3.reshape(1, 30))

    hd = jnp.concatenate([hd3.reshape(1300, F), _const16(1300)], axis=1)
    ipart = _sc_gs(W, 1408, 5, 80, "sc_info")(hd, nr3, nc3)
    (nsm,) = _tc_info("tc_info")(
        ipart[:, :1300, :F], xp3[:1300, :F], dinv0.reshape(1300, 1))

    return out, nsm.reshape(()), gemb


# speedup vs baseline: 43.1339x; 1.5354x over previous
import functools

import jax
import jax.numpy as jnp
from jax import lax
from jax.experimental import pallas as pl
from jax.experimental.pallas import tpu as pltpu
from jax.experimental.pallas import tpu_sc as plsc

F = 128
W = 144
G = 100
NC, NS, LN = 2, 16, 16
NW = NC * NS
E = 320000
BK = 80
EROWS = E // BK
NBLK = EROWS // NW
NBUF = 5
NEG = -3.4e38

f32 = jnp.float32
i32 = jnp.int32


def _mesh():
    return plsc.VectorSubcoreMesh(core_axis_name="c", subcore_axis_name="s")


def _ring(nblk, nbuf, bufs, g_start, g_wait, s_start, s_wait):
    g_start(0, 0)
    ngrp = -(-nblk // nbuf)

    @pl.loop(0, ngrp)
    def _(g):
        j0 = g * nbuf
        for u in range(nbuf):
            j = j0 + u

            @pl.when((j >= nbuf - 1) & (j < nblk))
            def _():
                s_wait((u + 1) % nbuf)

            @pl.when(j < nblk - 1)
            def _():
                g_start(j + 1, (u + 1) % nbuf)

            @pl.when(j < nblk)
            def _():
                g_wait(u)
                s_start(j, u)

    for d in range(nbuf - 1):
        s_wait((nblk - (nbuf - 1) + d) % nbuf)


def _sc_gs(w, ndp, nbuf, bk, name):
    rpt = ndp // NS
    erows = E // bk
    nblk = erows // NW

    @functools.partial(
        pl.kernel,
        out_type=jax.ShapeDtypeStruct((NC, ndp, w), f32),
        mesh=_mesh(),
        compiler_params=pltpu.CompilerParams(use_tc_tiling_on_sc=False, needs_layout_passes=False),
        scratch_types=[
            pltpu.VMEM_SHARED((ndp, w), f32),
            pltpu.VMEM((nblk, bk), i32),
            pltpu.VMEM((nblk, bk), i32),
        ] + [pltpu.VMEM((bk, w), f32) for _ in range(nbuf)] + [
            pltpu.VMEM((8, w), f32),
            pltpu.SemaphoreType.DMA,
            pltpu.SemaphoreType.DMA,
        ],
        name=name,
    )
    def k(src, nr, nc, out, *scr):
        acc, nrb, ncb = scr[0], scr[1], scr[2]
        bufs = scr[3:3 + nbuf]
        zbuf, gsem, ssem = scr[3 + nbuf], scr[4 + nbuf], scr[5 + nbuf]
        c = lax.axis_index("c")
        s = lax.axis_index("s")
        wid = c * NS + s
        pltpu.async_copy(nr.at[pl.ds(wid * nblk, nblk)], nrb, gsem)
        pltpu.async_copy(nc.at[pl.ds(wid * nblk, nblk)], ncb, gsem)
        for i in range(8):
            for q in range(w // LN):
                zbuf[i, pl.ds(q * LN, LN)] = jnp.zeros((LN,), f32)
        r0 = s * rpt

        @pl.loop(0, rpt // 8)
        def _(i):
            pltpu.async_copy(zbuf, acc.at[pl.ds(r0 + i * 8, 8)], ssem)

        @pl.loop(0, rpt // 8)
        def _(i):
            pltpu.make_async_copy(zbuf, acc.at[pl.ds(r0, 8)], ssem).wait()

        pltpu.make_async_copy(nr.at[pl.ds(0, nblk)], nrb, gsem).wait()
        pltpu.make_async_copy(nc.at[pl.ds(0, nblk)], ncb, gsem).wait()
        plsc.subcore_barrier()

        def g_start(j, u):
            pltpu.async_copy(src.at[nrb.at[j]], bufs[u], gsem)

        def g_wait(u):
            pltpu.make_async_copy(src.at[nrb.at[0]], bufs[u], gsem).wait()

        def s_start(j, u):
            pltpu.async_copy(bufs[u], acc.at[ncb.at[j]], ssem, add=True)

        def s_wait(u):
            pltpu.make_async_copy(bufs[u], acc.at[ncb.at[0]], ssem).wait()

        _ring(nblk, nbuf, bufs, g_start, g_wait, s_start, s_wait)
        plsc.subcore_barrier()
        pltpu.sync_copy(acc.at[pl.ds(r0, rpt)], out.at[c, pl.ds(r0, rpt)])

    return k


def _sc_remap(nm_rows, trash, perm_rows, ndp, do_conv, nbuf, tmask, name):
    w2 = W if do_conv else 16
    rpt = ndp // NS
    pb = perm_rows // NW

    scratch = [
        pltpu.VMEM_SHARED((ndp, w2), f32),
        pltpu.VMEM((nm_rows,), i32),
        pltpu.VMEM((NBLK, BK), i32),
        pltpu.VMEM((NBLK, BK), i32),
        pltpu.VMEM((NBLK, BK), i32),
        pltpu.VMEM((NBLK, BK), i32),
        pltpu.VMEM((8, 8) if do_conv else (NBLK, BK), i32),
    ] + [pltpu.VMEM((BK, W if do_conv else 16), f32) for _ in range(nbuf)] + [
        pltpu.VMEM((8, 8) if do_conv else (BK, W), f32),
        pltpu.VMEM((8, w2), f32),
        pltpu.VMEM((1, BK), i32),
        pltpu.SemaphoreType.DMA,
        pltpu.SemaphoreType.DMA,
    ]

    @functools.partial(
        pl.kernel,
        out_type=(
            jax.ShapeDtypeStruct((EROWS, BK), i32),
            jax.ShapeDtypeStruct((EROWS, BK), i32),
            jax.ShapeDtypeStruct((perm_rows * BK, W), f32),
            jax.ShapeDtypeStruct((NC, ndp, w2), f32),
        ),
        mesh=_mesh(),
        compiler_params=pltpu.CompilerParams(use_tc_tiling_on_sc=False, needs_layout_passes=False),
        scratch_types=scratch,
        name=name,
    )
    def k(hg, nr, nc, nm, perm, nrn, ncn, xp, part, *scr):
        acc, nmb, nrb, ncb, nrob, ncob, xb = scr[:7]
        bufs = scr[7:7 + nbuf]
        prow, zbuf, pbuf, gsem, ssem = scr[7 + nbuf:12 + nbuf]
        c = lax.axis_index("c")
        s = lax.axis_index("s")
        wid = c * NS + s
        pltpu.async_copy(nm, nmb, gsem)
        pltpu.async_copy(nr.at[pl.ds(wid * NBLK, NBLK)], nrb, gsem)
        pltpu.async_copy(nc.at[pl.ds(wid * NBLK, NBLK)], ncb, gsem)
        for i in range(8):
            for q in range(w2 // LN):
                zbuf[i, pl.ds(q * LN, LN)] = jnp.zeros((LN,), f32)
        r0 = s * rpt

        @pl.loop(0, rpt // 8)
        def _(i):
            pltpu.async_copy(zbuf, acc.at[pl.ds(r0 + i * 8, 8)], ssem)

        @pl.loop(0, rpt // 8)
        def _(i):
            pltpu.make_async_copy(zbuf, acc.at[pl.ds(r0, 8)], ssem).wait()

        pltpu.make_async_copy(nm, nmb, gsem).wait()
        pltpu.make_async_copy(nr.at[pl.ds(0, NBLK)], nrb, gsem).wait()
        pltpu.make_async_copy(nc.at[pl.ds(0, NBLK)], ncb, gsem).wait()
        if not do_conv:
            oh = jnp.where(lax.broadcasted_iota(i32, (LN,), 0) == 0,
                           jnp.float32(1), jnp.float32(0))
            for i in range(BK):
                bufs[0][i, pl.ds(0, 16)] = oh

        @pl.loop(0, NBLK)
        def _(j):
            for t in range(BK // LN):
                sl = pl.ds(t * LN, LN)
                nri = nrb[j, sl]
                a = plsc.load_gather(nmb, [nri])
                b = plsc.load_gather(nmb, [ncb[j, sl]])
                valid = (a >= 0) & (b >= 0)
                spread = jnp.where(nri >= trash, nri - trash, nri)
                nrob[j, sl] = jnp.where(valid, a, spread)
                tr = trash + (spread & tmask)
                if do_conv:
                    ncob[j, sl] = jnp.where(valid, b, tr)
                else:
                    ncob[j, sl] = jnp.where(valid, a, tr)
                    xb[j, sl] = jnp.where(valid, b, tr)

        pltpu.async_copy(nrob, nrn.at[pl.ds(wid * NBLK, NBLK)], gsem)
        pltpu.async_copy(ncob if do_conv else xb,
                         ncn.at[pl.ds(wid * NBLK, NBLK)], gsem)
        pltpu.make_async_copy(nrob, nrn.at[pl.ds(0, NBLK)], gsem).wait()
        pltpu.make_async_copy(ncob, ncn.at[pl.ds(0, NBLK)], gsem).wait()
        plsc.subcore_barrier()

        def g_start(j, u):
            pltpu.async_copy(hg.at[nrb.at[j]], bufs[u], gsem)

        def g_wait(u):
            pltpu.make_async_copy(hg.at[nrb.at[0]], bufs[u], gsem).wait()

        def s_start(j, u):
            pltpu.async_copy(bufs[u], acc.at[ncob.at[j]], ssem, add=True)

        def s_wait(u):
            pltpu.make_async_copy(bufs[u], acc.at[ncob.at[0]], ssem).wait()

        if do_conv:
            _ring(NBLK, nbuf, bufs, g_start, g_wait, s_start, s_wait)
        else:
            @pl.loop(0, NBLK)
            def _(j):
                @pl.when(j >= nbuf - 1)
                def _():
                    pltpu.make_async_copy(
                        bufs[0], acc.at[ncob.at[0]], ssem).wait()

                pltpu.async_copy(bufs[0], acc.at[ncob.at[j]], ssem,
                                 add=True)

            for d in range(nbuf - 1):
                pltpu.make_async_copy(bufs[0], acc.at[ncob.at[0]],
                                      ssem).wait()

        plsc.subcore_barrier()
        pltpu.sync_copy(acc.at[pl.ds(r0, rpt)], part.at[c, pl.ds(r0, rpt)])

        prow_ref = bufs[0] if do_conv else prow
        for r in range(pb):
            pr = wid * pb + r
            pltpu.sync_copy(perm.at[pl.ds(pr, 1)], pbuf)
            pltpu.sync_copy(hg.at[pbuf.at[0]], prow_ref)
            pltpu.sync_copy(prow_ref, xp.at[pl.ds(pr * BK, BK)])

    return k


def _leaky(x):
    return jnp.where(x >= 0, x, 0.1 * x)


def _tc_dense(rc, br, name):
    def body(part, xf, wl, bl, wr, wp, h_o, u16_o, hh_o, dinv_o):
        sfull = part[0] + part[1]
        cnt = sfull[:, 128:129]
        aggr = sfull[:, :F] / jnp.maximum(cnt, 1.0)
        z = (jnp.dot(aggr, wl[...], preferred_element_type=f32)
             + bl[...]
             + jnp.dot(xf[...], wr[...], preferred_element_type=f32))
        h = _leaky(z)
        dinv = lax.rsqrt(cnt + 1.0)
        hh = jnp.dot(h, wp[...], preferred_element_type=f32)
        u = hh * dinv
        lane = lax.broadcasted_iota(i32, (br, 16), 1)
        h_o[...] = h
        u16_o[...] = jnp.where(lane == 0, u, 0.0)
        hh_o[...] = hh
        dinv_o[...] = dinv

    return pl.pallas_call(
        body,
        grid=(rc // br,),
        in_specs=[
            pl.BlockSpec((2, br, W), lambda i: (0, i, 0)),
            pl.BlockSpec((br, F), lambda i: (i, 0)),
            pl.BlockSpec((F, F), lambda i: (0, 0)),
            pl.BlockSpec((1, F), lambda i: (0, 0)),
            pl.BlockSpec((F, F), lambda i: (0, 0)),
            pl.BlockSpec((F, 1), lambda i: (0, 0)),
        ],
        out_specs=[
            pl.BlockSpec((br, F), lambda i: (i, 0)),
            pl.BlockSpec((br, 16), lambda i: (i, 0)),
            pl.BlockSpec((br, 1), lambda i: (i, 0)),
            pl.BlockSpec((br, 1), lambda i: (i, 0)),
        ],
        out_shape=[
            jax.ShapeDtypeStruct((rc, F), f32),
            jax.ShapeDtypeStruct((rc, 16), f32),
            jax.ShapeDtypeStruct((rc, 1), f32),
            jax.ShapeDtypeStruct((rc, 1), f32),
        ],
        name=name,
    )


def _tc_pool(n_per, kk, name):
    def body(spA, hh, dinv, h3d, bp, nm_o, perm_o, hg_o):
        sp = spA[0] + spA[1]
        d = dinv[...]
        score = sp * d + hh[...] * d * d + bp[0, 0]
        work = score
        iota = lax.broadcasted_iota(i32, (G, n_per), 1)
        rowid = lax.broadcasted_iota(i32, (G, n_per), 0)
        iota_k = lax.broadcasted_iota(i32, (G, kk), 1)
        rowid_k = lax.broadcasted_iota(i32, (G, kk), 0)
        nm2d = jnp.full((G, n_per), -1, i32)
        perm = jnp.zeros((G, kk), i32)
        for m in range(kk):
            mx = jnp.max(work, axis=1, keepdims=True)
            eq = work == mx
            jf = jnp.min(jnp.where(eq, iota, 1 << 30), axis=1, keepdims=True)
            oh = iota == jf
            nm2d = jnp.where(oh, rowid * kk + m, nm2d)
            perm = jnp.where(iota_k == m, jf + rowid_k * n_per, perm)
            work = jnp.where(oh, NEG, work)
        gate = jnp.tanh(score)
        nm_o[...] = nm2d
        perm_o[...] = perm
        hg_o[...] = h3d[...] * gate[:, :, None]

    return pl.pallas_call(
        body,
        out_shape=[
            jax.ShapeDtypeStruct((G, n_per), i32),
            jax.ShapeDtypeStruct((G, kk), i32),
            jax.ShapeDtypeStruct((G, n_per, F), f32),
        ],
        name=name,
    )


def _readout(xr, kk):
    mx = xr[:, 0, :]
    sm = xr[:, 0, :]
    for i in range(1, kk):
        v = xr[:, i, :]
        mx = jnp.maximum(mx, v)
        sm = sm + v
    return jnp.concatenate([mx, sm * (1.0 / kk)], axis=1)


def _tc_head(name):
    def body(x1r, x2r, x3r, degA, wf1, bf1, wf2, bf2, wf3, bf3,
             out_o, g_o, hd_o, dinv_o):
        x1h = _readout(x1r, 50)
        x2h = _readout(x2r, 25)
        x3h = _readout(x3r, 13)
        g = _leaky(x1h) + _leaky(x2h) + _leaky(x3h)
        o = _leaky(jnp.dot(g, wf1[...], preferred_element_type=f32) + bf1[...])
        o = _leaky(jnp.dot(o, wf2[...], preferred_element_type=f32) + bf2[...])
        o = jnp.dot(o, wf3[...], preferred_element_type=f32) + bf3[...]
        deg = degA[0] + degA[1]
        dinv0 = jnp.where(deg > 0,
                          lax.rsqrt(jnp.maximum(deg, 1e-12)), 0.0)
        out_o[...] = o
        g_o[...] = g
        hd_o[...] = x3r[...] * dinv0[:, :, None]
        dinv_o[...] = dinv0

    return pl.pallas_call(
        body,
        out_shape=[
            jax.ShapeDtypeStruct((G, 30), f32),
            jax.ShapeDtypeStruct((G, 256), f32),
            jax.ShapeDtypeStruct((G, 13, F), f32),
            jax.ShapeDtypeStruct((G, 13), f32),
        ],
        name=name,
    )


def _tc_info(name):
    def body(ip, x3f, dinv0, out_o):
        agg = (ip[0] + ip[1]) * dinv0[...]
        info = x3f[...] - agg
        ns = jnp.sum(jnp.abs(info), axis=1, keepdims=True)
        out_o[...] = jnp.sum(ns).reshape(1, 1) * (1.0 / 1300.0)

    return pl.pallas_call(
        body,
        out_shape=[jax.ShapeDtypeStruct((1, 1), f32)],
        name=name,
    )


def _const16(n):
    return jnp.concatenate([jnp.ones((n, 1), f32), jnp.zeros((n, 15), f32)],
                           axis=1)


def kernel(x, edge_index, batch, Wl1, bl1, Wr1, Wl2, bl2, Wr2, Wl3, bl3, Wr3,
           Wp1, bp1, Wp2, bp2, Wp3, bp3, Wf1, bf1, Wf2, bf2, Wf3, bf3):
    del batch
    nr0 = edge_index[0].reshape(EROWS, BK)
    nc0 = edge_index[1].reshape(EROWS, BK)
    nr0c = edge_index[0].reshape(E // 50, 50)
    nc0c = edge_index[1].reshape(E // 50, 50)
    hg0 = jnp.concatenate([x, _const16(10000)], axis=1)

    part1 = _sc_gs(W, 10112, 2, 50, "sc_conv1")(hg0, nr0c, nc0c)
    h1, u16_1, hh1, dinv1 = _tc_dense(10000, 1000, "tc_dense1")(
        part1[:, :10000], x, Wl1, bl1.reshape(1, F), Wr1, Wp1)
    sp1 = _sc_gs(16, 10112, 5, 80, "sc_score1")(u16_1, nr0, nc0)
    nm1_2d, perm1_2d, hg1_3d = _tc_pool(100, 50, "tc_pool1")(
        sp1[:, :10000, 0].reshape(2, G, 100),
        hh1[:, 0].reshape(G, 100),
        dinv1[:, 0].reshape(G, 100),
        h1.reshape(G, 100, F),
        bp1.reshape(1, 1))
    nm1 = nm1_2d.reshape(10000)
    perm1 = jnp.concatenate(
        [perm1_2d.reshape(5000), jnp.zeros((120,), i32)]).reshape(64, BK)
    hg1 = jnp.concatenate([hg1_3d.reshape(10000, F), _const16(10000)], axis=1)

    nr1, nc1, xp1, part2 = _sc_remap(
        10000, 5000, 64, 5120, True, 2, 63, "sc_remap_conv2")(hg1, nr0, nc0, nm1,
                                                       perm1)
    h2, u16_2, hh2, dinv2 = _tc_dense(5120, 640, "tc_dense2")(
        part2, xp1[:, :F], Wl2, bl2.reshape(1, F), Wr2, Wp2)
    sp2 = _sc_gs(16, 5120, 5, 80, "sc_score2")(u16_2, nr1, nc1)
    nm2_2d, perm2_2d, hg2_3d = _tc_pool(50, 25, "tc_pool2")(
        sp2[:, :5000, 0].reshape(2, G, 50),
        hh2[:5000, 0].reshape(G, 50),
        dinv2[:5000, 0].reshape(G, 50),
        h2[:5000].reshape(G, 50, F),
        bp2.reshape(1, 1))
    nm2 = jnp.concatenate([nm2_2d.reshape(5000), jnp.full((64,), -1, i32)])
    perm2 = jnp.concatenate(
        [perm2_2d.reshape(2500), jnp.zeros((60,), i32)]).reshape(32, BK)
    hg2 = jnp.concatenate([hg2_3d.reshape(5000, F), _const16(5000)], axis=1)

    nr2, nc2, xp2, part3 = _sc_remap(
        5064, 2500, 32, 2560, True, 5, 31, "sc_remap_conv3")(hg2, nr1, nc1, nm2,
                                                      perm2)
    h3, u16_3, hh3, dinv3 = _tc_dense(2560, 640, "tc_dense3")(
        part3, xp2[:, :F], Wl3, bl3.reshape(1, F), Wr3, Wp3)
    sp3 = _sc_gs(16, 2560, 5, 80, "sc_score3")(u16_3, nr2, nc2)
    nm3_2d, perm3_2d, hg3_3d = _tc_pool(25, 13, "tc_pool3")(
        sp3[:, :2500, 0].reshape(2, G, 25),
        hh3[:2500, 0].reshape(G, 25),
        dinv3[:2500, 0].reshape(G, 25),
        h3[:2500].reshape(G, 25, F),
        bp3.reshape(1, 1))
    nm3 = jnp.concatenate([nm3_2d.reshape(2500), jnp.full((36,), -1, i32)])
    perm3 = jnp.concatenate(
        [perm3_2d.reshape(1300), jnp.zeros((1260,), i32)]).reshape(32, BK)
    hg3 = jnp.concatenate([hg3_3d.reshape(2500, F), _const16(2500)], axis=1)

    nr3, nc3, xp3, degp = _sc_remap(
        2536, 1300, 32, 1408, False, 5, 31, "sc_remap_pool4")(hg3, nr2, nc2, nm3,
                                                       perm3)

    out, gemb, hd3, dinv0 = _tc_head("tc_head")(
        xp1[:5000, :F].reshape(G, 50, F),
        xp2[:2500, :F].reshape(G, 25, F),
        xp3[:1300, :F].reshape(G, 13, F),
        degp[:, :1300, 0].reshape(2, G, 13),
        Wf1, bf1.reshape(1, F), Wf2, bf2.reshape(1, 64), Wf3,
        bf3.reshape(1, 30))

    hd = jnp.concatenate([hd3.reshape(1300, F), _const16(1300)], axis=1)
    ipart = _sc_gs(W, 1408, 5, 80, "sc_info")(hd, nr3, nc3)
    (nsm,) = _tc_info("tc_info")(
        ipart[:, :1300, :F], xp3[:1300, :F], dinv0.reshape(1300, 1))

    return out, nsm.reshape(()), gemb
